# baseline (speedup 1.0000x reference)
import jax
import jax.numpy as jnp
from jax.experimental import pallas as pl

_INTERPRET = False

E = 64
K = 8
F = 64
T = 2
OUT = 10
BB = 128


def _moe_block(x_ref, Wb_ref, bb_ref, Wer_ref, be_ref, Wp_ref, Wgr_ref,
               bg_ref, Wh_ref, bh_ref, Rg_ref, Rw_ref, M1_ref, M2_ref,
               o0_ref, o1_ref):
    n = x_ref.shape[0]
    x = x_ref[...]
    f32 = jnp.float32
    hi = jax.lax.Precision.HIGHEST

    L = jnp.dot(x, Wp_ref[...], preferred_element_type=f32)
    m = jnp.max(L, axis=-1, keepdims=True)
    eL = jnp.exp(L - m)
    denom = jnp.dot(eL, M1_ref[...], preferred_element_type=f32, precision=hi)
    gall = jnp.dot(x, Wgr_ref[...], preferred_element_type=f32) + bg_ref[...]
    gps = []
    for t in range(T):
        g = gall[:, t * E:(t + 1) * E]
        g_t = jnp.dot(g, Rg_ref[...], preferred_element_type=f32, precision=hi)
        num = jnp.dot(eL * g_t, M1_ref[...], preferred_element_type=f32,
                      precision=hi)
        gps.append(num / denom)

    h1 = jnp.maximum(jnp.dot(x, Wb_ref[...], preferred_element_type=f32)
                     + bb_ref[...], 0.0)
    h2 = jnp.dot(h1, Wer_ref[...], preferred_element_type=f32) + be_ref[...]

    iota = jax.lax.broadcasted_iota(jnp.int32, (n, E), 1)
    for t, o_ref in ((0, o0_ref), (1, o1_ref)):
        gp = gps[t]
        sel = jnp.zeros((n, E), dtype=jnp.bool_)
        for _ in range(K):
            cur = jnp.where(sel, -jnp.inf, gp)
            mk = jnp.max(cur, axis=-1, keepdims=True)
            first = jnp.min(jnp.where(cur == mk, iota, E), axis=-1, keepdims=True)
            sel = sel | (iota == first)
        mx = jnp.max(gp, axis=-1, keepdims=True)
        ex = jnp.where(sel, jnp.exp(gp - mx), 0.0)
        w = ex / jnp.sum(ex, axis=-1, keepdims=True)
        w_t = jnp.dot(w, Rw_ref[...], preferred_element_type=f32)
        comb = jnp.dot(h2 * w_t, M2_ref[...], preferred_element_type=f32)
        o_ref[...] = jnp.dot(comb, Wh_ref[t], preferred_element_type=f32) \
            + bh_ref[t][None, :]


def kernel(x, W_bottom, b_bottom, W_experts, b_experts, W_perm, W_gate,
           b_gate, W_head, b_head):
    B, D = x.shape
    Wer = W_experts.transpose(1, 0, 2).reshape(D, E * F)
    Wgr = W_gate.transpose(1, 0, 2).reshape(D, T * E)
    bb2 = b_bottom.reshape(1, D)
    bg2 = b_gate.reshape(1, T * E)
    be2 = b_experts.reshape(1, E * F)
    eye = jnp.eye(E, dtype=jnp.float32)
    Rg = jnp.tile(eye, (1, E))
    Rw = jnp.repeat(eye, F, axis=1)
    M1 = jnp.repeat(eye, E, axis=0)
    M2 = jnp.tile(eye, (E, 1))
    grid = (B // BB,)
    full = lambda shape: pl.BlockSpec(shape, lambda i: (0,) * len(shape))
    o0, o1 = pl.pallas_call(
        _moe_block,
        grid=grid,
        in_specs=[
            pl.BlockSpec((BB, D), lambda i: (i, 0)),
            full((D, D)),
            full((1, D)),
            full((D, E * F)),
            full((1, E * F)),
            full((D, E * E)),
            full((D, T * E)),
            full((1, T * E)),
            full((T, F, OUT)),
            full((T, OUT)),
            full((E, E * E)),
            full((E, E * F)),
            full((E * E, E)),
            full((E * F, F)),
        ],
        out_specs=[pl.BlockSpec((BB, OUT), lambda i: (i, 0)),
                   pl.BlockSpec((BB, OUT), lambda i: (i, 0))],
        out_shape=[jax.ShapeDtypeStruct((B, OUT), jnp.float32),
                   jax.ShapeDtypeStruct((B, OUT), jnp.float32)],
        interpret=_INTERPRET,
    )(x, W_bottom, bb2, Wer, be2, W_perm, Wgr, bg2, W_head, b_head,
      Rg, Rw, M1, M2)
    return (o0, o1)


# speedup vs baseline: 1.2577x; 1.2577x over previous
import jax
import jax.numpy as jnp
from jax.experimental import pallas as pl

_INTERPRET = False

E = 64
K = 8
F = 64
T = 2
OUT = 10
BB = 128


def _sel_dot(a, Mb, passes=3):
    f32 = jnp.float32
    acc = None
    r = a
    for _ in range(passes):
        ab = r.astype(jnp.bfloat16)
        r = r - ab.astype(f32)
        p = jnp.dot(ab, Mb, preferred_element_type=f32)
        acc = p if acc is None else acc + p
    return acc


def _moe_block(x_ref, Wb_ref, bb_ref, Wer_ref, be_ref, Wp_ref, Wgr_ref,
               bg_ref, Wh_ref, bh_ref, Rg_ref, Rw_ref, M1_ref, M2_ref,
               o0_ref, o1_ref):
    n = x_ref.shape[0]
    x = x_ref[...]
    f32 = jnp.float32

    L = jnp.dot(x, Wp_ref[...], preferred_element_type=f32)
    m = jnp.max(L, axis=-1, keepdims=True)
    eL = jnp.exp(L - m)
    den = _sel_dot(eL, M1_ref[...])
    gall = jnp.dot(x, Wgr_ref[...], preferred_element_type=f32) + bg_ref[...]
    gps = []
    for t in range(T):
        g = gall[:, t * E:(t + 1) * E]
        g_t = _sel_dot(g, Rg_ref[...])
        num = _sel_dot(eL * g_t, M1_ref[...])
        gps.append(num / den)

    h1 = jnp.maximum(jnp.dot(x, Wb_ref[...], preferred_element_type=f32)
                     + bb_ref[...], 0.0)
    h2 = jnp.dot(h1, Wer_ref[...], preferred_element_type=f32) + be_ref[...]

    iota = jax.lax.broadcasted_iota(jnp.int32, (E, n), 0)
    for t, o_ref in ((0, o0_ref), (1, o1_ref)):
        gpT = gps[t].T
        sel = jnp.zeros((E, n), dtype=jnp.bool_)
        for _ in range(K):
            cur = jnp.where(sel, -jnp.inf, gpT)
            mk = jnp.max(cur, axis=0, keepdims=True)
            first = jnp.min(jnp.where(cur == mk, iota, E), axis=0, keepdims=True)
            sel = sel | (iota == first)
        mx = jnp.max(gpT, axis=0, keepdims=True)
        ex = jnp.where(sel, jnp.exp(gpT - mx), 0.0)
        w = (ex / jnp.sum(ex, axis=0, keepdims=True)).T
        w_t = _sel_dot(w, Rw_ref[...], passes=2)
        comb = _sel_dot(h2 * w_t, M2_ref[...], passes=2)
        o_ref[...] = jnp.dot(comb, Wh_ref[t], preferred_element_type=f32) \
            + bh_ref[t][None, :]


def kernel(x, W_bottom, b_bottom, W_experts, b_experts, W_perm, W_gate,
           b_gate, W_head, b_head):
    B, D = x.shape
    Wer = W_experts.transpose(1, 0, 2).reshape(D, E * F)
    Wgr = W_gate.transpose(1, 0, 2).reshape(D, T * E)
    bb2 = b_bottom.reshape(1, D)
    bg2 = b_gate.reshape(1, T * E)
    be2 = b_experts.reshape(1, E * F)
    eye = jnp.eye(E, dtype=jnp.bfloat16)
    Rg = jnp.tile(eye, (1, E))
    Rw = jnp.repeat(eye, F, axis=1)
    M1 = jnp.repeat(eye, E, axis=0)
    M2 = jnp.tile(eye, (E, 1))
    grid = (B // BB,)
    full = lambda shape: pl.BlockSpec(shape, lambda i: (0,) * len(shape))
    o0, o1 = pl.pallas_call(
        _moe_block,
        grid=grid,
        in_specs=[
            pl.BlockSpec((BB, D), lambda i: (i, 0)),
            full((D, D)),
            full((1, D)),
            full((D, E * F)),
            full((1, E * F)),
            full((D, E * E)),
            full((D, T * E)),
            full((1, T * E)),
            full((T, F, OUT)),
            full((T, OUT)),
            full((E, E * E)),
            full((E, E * F)),
            full((E * E, E)),
            full((E * F, F)),
        ],
        out_specs=[pl.BlockSpec((BB, OUT), lambda i: (i, 0)),
                   pl.BlockSpec((BB, OUT), lambda i: (i, 0))],
        out_shape=[jax.ShapeDtypeStruct((B, OUT), jnp.float32),
                   jax.ShapeDtypeStruct((B, OUT), jnp.float32)],
        interpret=_INTERPRET,
    )(x, W_bottom, bb2, Wer, be2, W_perm, Wgr, bg2, W_head, b_head,
      Rg, Rw, M1, M2)
    return (o0, o1)


# speedup vs baseline: 1.5546x; 1.2361x over previous
import jax
import jax.numpy as jnp
from jax.experimental import pallas as pl

_INTERPRET = False

E = 64
K = 8
F = 64
T = 2
OUT = 10
BB = 128


def _sel_dot(a, Mb, passes=3):
    f32 = jnp.float32
    acc = None
    r = a
    for _ in range(passes):
        ab = r.astype(jnp.bfloat16)
        r = r - ab.astype(f32)
        p = jnp.dot(ab, Mb, preferred_element_type=f32)
        acc = p if acc is None else acc + p
    return acc


def _moe_block(x_ref, Wb_ref, bb_ref, Wer_ref, be_ref, Wp_ref, Wgr_ref,
               bg_ref, Wh_ref, bh_ref, M1_ref, o0_ref, o1_ref):
    n = x_ref.shape[0]
    x = x_ref[...]
    f32 = jnp.float32

    L = jnp.dot(x, Wp_ref[...], preferred_element_type=f32)
    m = jnp.max(L, axis=-1, keepdims=True)
    eL = jnp.exp(L - m)
    den = _sel_dot(eL, M1_ref[...])
    gall = jnp.dot(x, Wgr_ref[...], preferred_element_type=f32) + bg_ref[...]
    gps = []
    for t in range(T):
        g = gall[:, t * E:(t + 1) * E]
        num = _sel_dot(eL * jnp.tile(g, (1, E)), M1_ref[...])
        gps.append(num / den)

    h1 = jnp.maximum(jnp.dot(x, Wb_ref[...], preferred_element_type=f32)
                     + bb_ref[...], 0.0)
    h2 = jnp.dot(h1, Wer_ref[...], preferred_element_type=f32) + be_ref[...]

    iota = jax.lax.broadcasted_iota(jnp.int32, (E, n), 0)
    for t, o_ref in ((0, o0_ref), (1, o1_ref)):
        gpT = gps[t].T
        sel = jnp.zeros((E, n), dtype=jnp.bool_)
        for _ in range(K):
            cur = jnp.where(sel, -jnp.inf, gpT)
            mk = jnp.max(cur, axis=0, keepdims=True)
            first = jnp.min(jnp.where(cur == mk, iota, E), axis=0, keepdims=True)
            sel = sel | (iota == first)
        mx = jnp.max(gpT, axis=0, keepdims=True)
        ex = jnp.where(sel, jnp.exp(gpT - mx), 0.0)
        w = (ex / jnp.sum(ex, axis=0, keepdims=True)).T
        comb = _sel_dot(h2 * jnp.tile(w, (1, F)), M1_ref[...], passes=2)
        o_ref[...] = jnp.dot(comb, Wh_ref[t], preferred_element_type=f32) \
            + bh_ref[t][None, :]


def kernel(x, W_bottom, b_bottom, W_experts, b_experts, W_perm, W_gate,
           b_gate, W_head, b_head):
    B, D = x.shape
    Werf = W_experts.transpose(1, 2, 0).reshape(D, F * E)
    Wgr = W_gate.transpose(1, 0, 2).reshape(D, T * E)
    bb2 = b_bottom.reshape(1, D)
    bg2 = b_gate.reshape(1, T * E)
    bef = b_experts.T.reshape(1, F * E)
    M1 = jnp.repeat(jnp.eye(E, dtype=jnp.bfloat16), E, axis=0)
    grid = (B // BB,)
    full = lambda shape: pl.BlockSpec(shape, lambda i: (0,) * len(shape))
    o0, o1 = pl.pallas_call(
        _moe_block,
        grid=grid,
        in_specs=[
            pl.BlockSpec((BB, D), lambda i: (i, 0)),
            full((D, D)),
            full((1, D)),
            full((D, F * E)),
            full((1, F * E)),
            full((D, E * E)),
            full((D, T * E)),
            full((1, T * E)),
            full((T, F, OUT)),
            full((T, OUT)),
            full((E * E, E)),
        ],
        out_specs=[pl.BlockSpec((BB, OUT), lambda i: (i, 0)),
                   pl.BlockSpec((BB, OUT), lambda i: (i, 0))],
        out_shape=[jax.ShapeDtypeStruct((B, OUT), jnp.float32),
                   jax.ShapeDtypeStruct((B, OUT), jnp.float32)],
        interpret=_INTERPRET,
    )(x, W_bottom, bb2, Werf, bef, W_perm, Wgr, bg2, W_head, b_head, M1)
    return (o0, o1)


# speedup vs baseline: 1.7277x; 1.1113x over previous
import jax
import jax.numpy as jnp
from jax.experimental import pallas as pl

_INTERPRET = False

E = 64
K = 8
F = 64
T = 2
OUT = 10
BB = 256


def _sel_dot(a, Mb, passes=3):
    f32 = jnp.float32
    acc = None
    r = a
    for _ in range(passes):
        ab = r.astype(jnp.bfloat16)
        r = r - ab.astype(f32)
        p = jnp.dot(ab, Mb, preferred_element_type=f32)
        acc = p if acc is None else acc + p
    return acc


def _moe_block(x_ref, Wb_ref, bb_ref, Wer_ref, be_ref, Wp_ref, Wgr_ref,
               bg_ref, Wh_ref, bh_ref, M1_ref, o0_ref, o1_ref):
    n = x_ref.shape[0]
    x = x_ref[...]
    f32 = jnp.float32

    L = jnp.dot(x, Wp_ref[...], preferred_element_type=f32)
    m = jnp.max(L, axis=-1, keepdims=True)
    eL = jnp.exp(L - m)
    den = _sel_dot(eL, M1_ref[...])
    gall = jnp.dot(x, Wgr_ref[...], preferred_element_type=f32) + bg_ref[...]
    gps = []
    for t in range(T):
        g = gall[:, t * E:(t + 1) * E]
        num = _sel_dot(eL * jnp.tile(g, (1, E)), M1_ref[...])
        gps.append(num / den)

    h1 = jnp.maximum(jnp.dot(x, Wb_ref[...], preferred_element_type=f32)
                     + bb_ref[...], 0.0)
    h2 = jnp.dot(h1, Wer_ref[...], preferred_element_type=f32) + be_ref[...]

    iota = jax.lax.broadcasted_iota(jnp.int32, (E, n), 0)
    for t, o_ref in ((0, o0_ref), (1, o1_ref)):
        gpT = gps[t].T
        sel = jnp.zeros((E, n), dtype=jnp.bool_)
        for _ in range(K):
            cur = jnp.where(sel, -jnp.inf, gpT)
            mk = jnp.max(cur, axis=0, keepdims=True)
            first = jnp.min(jnp.where(cur == mk, iota, E), axis=0, keepdims=True)
            sel = sel | (iota == first)
        mx = jnp.max(gpT, axis=0, keepdims=True)
        ex = jnp.where(sel, jnp.exp(gpT - mx), 0.0)
        w = (ex / jnp.sum(ex, axis=0, keepdims=True)).T
        comb = _sel_dot(h2 * jnp.tile(w, (1, F)), M1_ref[...], passes=2)
        o_ref[...] = jnp.dot(comb, Wh_ref[t], preferred_element_type=f32) \
            + bh_ref[t][None, :]


def kernel(x, W_bottom, b_bottom, W_experts, b_experts, W_perm, W_gate,
           b_gate, W_head, b_head):
    B, D = x.shape
    Werf = W_experts.transpose(1, 2, 0).reshape(D, F * E)
    Wgr = W_gate.transpose(1, 0, 2).reshape(D, T * E)
    bb2 = b_bottom.reshape(1, D)
    bg2 = b_gate.reshape(1, T * E)
    bef = b_experts.T.reshape(1, F * E)
    M1 = jnp.repeat(jnp.eye(E, dtype=jnp.bfloat16), E, axis=0)
    grid = (B // BB,)
    full = lambda shape: pl.BlockSpec(shape, lambda i: (0,) * len(shape))
    o0, o1 = pl.pallas_call(
        _moe_block,
        grid=grid,
        in_specs=[
            pl.BlockSpec((BB, D), lambda i: (i, 0)),
            full((D, D)),
            full((1, D)),
            full((D, F * E)),
            full((1, F * E)),
            full((D, E * E)),
            full((D, T * E)),
            full((1, T * E)),
            full((T, F, OUT)),
            full((T, OUT)),
            full((E * E, E)),
        ],
        out_specs=[pl.BlockSpec((BB, OUT), lambda i: (i, 0)),
                   pl.BlockSpec((BB, OUT), lambda i: (i, 0))],
        out_shape=[jax.ShapeDtypeStruct((B, OUT), jnp.float32),
                   jax.ShapeDtypeStruct((B, OUT), jnp.float32)],
        interpret=_INTERPRET,
    )(x, W_bottom, bb2, Werf, bef, W_perm, Wgr, bg2, W_head, b_head, M1)
    return (o0, o1)
